# baseline (speedup 1.0000x reference)
import functools

import jax
import jax.numpy as jnp
from jax import lax
from jax.experimental import pallas as pl
from jax.experimental.pallas import tpu as pltpu
from jax.experimental.pallas import tpu_sc as plsc

D = 64
K = 1024
BETA = 0.25

ROWS = 16 * 1024
BLOCK_ROWS = 1024
NUM_BLOCKS = ROWS // BLOCK_ROWS

NUM_WORKERS = 32
BPW = ROWS // NUM_WORKERS
CHUNK = 128
NCHUNKS = BPW // CHUNK


def _assign_body(z_ref, e_ref, idx_ref, loss_ref, enb_ref, ids_ref):
    i = pl.program_id(0)
    e = e_ref[...]

    @pl.when(i == 0)
    def _prep():
        en = jnp.sum(e * e, axis=1)
        enb_ref[...] = jnp.broadcast_to(en[:, None], (K, BLOCK_ROWS))
        ids_ref[...] = lax.broadcasted_iota(
            jnp.int32, (K, BLOCK_ROWS), 0).astype(jnp.float32)

    z = z_ref[0]
    z2 = z * 2.0
    s2 = lax.dot_general(e, z2, (((1,), (0,)), ((), ())),
                         preferred_element_type=jnp.float32)
    zn = jnp.sum(z * z, axis=0)
    d = (zn[None, :] + enb_ref[...]) - s2
    col_min = jnp.min(d, axis=0)
    ids = ids_ref[...]
    idx_f = jnp.min(jnp.where(d == col_min[None, :], ids, float(K)), axis=0)
    idx_ref[0, 0, :] = idx_f.astype(jnp.int32)
    partial = jnp.sum(col_min)

    @pl.when(i == 0)
    def _init():
        loss_ref[0, 0] = partial

    @pl.when(i != 0)
    def _acc():
        loss_ref[0, 0] += partial


def _assign(z_t, embeddings):
    return pl.pallas_call(
        _assign_body,
        grid=(NUM_BLOCKS,),
        in_specs=[
            pl.BlockSpec((1, D, BLOCK_ROWS), lambda i: (i, 0, 0)),
            pl.BlockSpec((K, D), lambda i: (0, 0)),
        ],
        out_specs=[
            pl.BlockSpec((1, 1, BLOCK_ROWS), lambda i: (i, 0, 0)),
            pl.BlockSpec((1, 1), lambda i: (0, 0), memory_space=pltpu.SMEM),
        ],
        out_shape=[
            jax.ShapeDtypeStruct((NUM_BLOCKS, 1, BLOCK_ROWS), jnp.int32),
            jax.ShapeDtypeStruct((1, 1), jnp.float32),
        ],
        scratch_shapes=[
            pltpu.VMEM((K, BLOCK_ROWS), jnp.float32),
            pltpu.VMEM((K, BLOCK_ROWS), jnp.float32),
        ],
    )(z_t, embeddings)


def _gather_body(et_hbm, idx_hbm, out_hbm, et_v, idx_v, tv):
    wid = lax.axis_index("s") * 2 + lax.axis_index("c")
    base = wid * BPW
    b = base // 1024
    off = base % 1024
    pltpu.sync_copy(et_hbm, et_v)
    pltpu.sync_copy(idx_hbm.at[pl.ds(base, BPW)], idx_v)

    @plsc.parallel_loop(0, BPW // 16, unroll=8)
    def _transpose(jj):
        idxv = idx_v[pl.ds(jj * 16, 16)]
        for d_ in range(D):
            row = jnp.full((16,), d_, jnp.int32)
            tv[d_, pl.ds(jj * 16, 16)] = plsc.load_gather(et_v, [row, idxv])
    pltpu.sync_copy(tv, out_hbm.at[b].at[:, pl.ds(off, 512)])


@functools.cache
def _gather():
    mesh = plsc.VectorSubcoreMesh(core_axis_name="c", subcore_axis_name="s")
    return pl.kernel(
        _gather_body,
        out_type=jax.ShapeDtypeStruct((16, D, 1024), jnp.float32),
        mesh=mesh,
        scratch_types=[
            pltpu.VMEM((D, K), jnp.float32),
            pltpu.VMEM((BPW,), jnp.int32),
            pltpu.VMEM((D, BPW), jnp.float32),
        ],
        compiler_params=pltpu.CompilerParams(needs_layout_passes=False),
    )


def kernel(z_e, embeddings):
    z_t = jnp.transpose(z_e, (0, 2, 1))
    idx3, loss_sum = _assign(z_t, embeddings)
    idx = idx3.reshape(ROWS)
    emb_t = jnp.transpose(embeddings)
    zq_t = _gather()(emb_t, idx)
    z_q_st = jnp.transpose(zq_t, (0, 2, 1))
    vq_loss = loss_sum[0, 0] * ((1.0 + BETA) / float(ROWS * D))
    return z_q_st, vq_loss


# speedup vs baseline: 1.1814x; 1.1814x over previous
import functools

import jax
import jax.numpy as jnp
from jax import lax
from jax.experimental import pallas as pl
from jax.experimental.pallas import tpu as pltpu
from jax.experimental.pallas import tpu_sc as plsc

D = 64
K = 1024
BETA = 0.25

ROWS = 16 * 1024
BLOCK_ROWS = 1024
NUM_BLOCKS = ROWS // BLOCK_ROWS

NUM_WORKERS = 32
BPW = ROWS // NUM_WORKERS
CHUNK = 128
NCHUNKS = BPW // CHUNK


def _assign_body(z_ref, e_ref, idx_ref, loss_ref, enb_ref, ids_ref):
    i = pl.program_id(0)
    e = e_ref[...]

    @pl.when(i == 0)
    def _prep():
        en = jnp.sum(e * e, axis=1)
        enb_ref[...] = jnp.broadcast_to(en[:, None], (K, BLOCK_ROWS))
        ids_ref[...] = lax.broadcasted_iota(
            jnp.int32, (K, BLOCK_ROWS), 0).astype(jnp.float32)

    z = z_ref[0]
    z2 = z * 2.0
    s2 = lax.dot_general(e, z2, (((1,), (0,)), ((), ())),
                         preferred_element_type=jnp.float32)
    zn = jnp.sum(z * z, axis=0)
    d = (zn[None, :] + enb_ref[...]) - s2
    col_min = jnp.min(d, axis=0)
    ids = ids_ref[...]
    idx_f = jnp.min(jnp.where(d == col_min[None, :], ids, float(K)), axis=0)
    idx_ref[0, 0, :] = idx_f.astype(jnp.int32)
    partial = jnp.sum(col_min)

    @pl.when(i == 0)
    def _init():
        loss_ref[0, 0] = partial

    @pl.when(i != 0)
    def _acc():
        loss_ref[0, 0] += partial


def _assign(z_t, embeddings):
    return pl.pallas_call(
        _assign_body,
        grid=(NUM_BLOCKS,),
        in_specs=[
            pl.BlockSpec((1, D, BLOCK_ROWS), lambda i: (i, 0, 0)),
            pl.BlockSpec((K, D), lambda i: (0, 0)),
        ],
        out_specs=[
            pl.BlockSpec((1, 1, BLOCK_ROWS), lambda i: (i, 0, 0)),
            pl.BlockSpec((1, 1), lambda i: (0, 0), memory_space=pltpu.SMEM),
        ],
        out_shape=[
            jax.ShapeDtypeStruct((NUM_BLOCKS, 1, BLOCK_ROWS), jnp.int32),
            jax.ShapeDtypeStruct((1, 1), jnp.float32),
        ],
        scratch_shapes=[
            pltpu.VMEM((K, BLOCK_ROWS), jnp.float32),
            pltpu.VMEM((K, BLOCK_ROWS), jnp.float32),
        ],
    )(z_t, embeddings)


_DG = 8
_PG = ROWS // (NUM_WORKERS // _DG)


def _gather_body(et_hbm, idx_hbm, out_hbm, et_v, idx_v, tv):
    wid = lax.axis_index("s") * 2 + lax.axis_index("c")
    dg = wid % _DG
    pg = wid // _DG
    pltpu.sync_copy(et_hbm.at[pl.ds(dg * _DG, _DG)], et_v)
    pltpu.sync_copy(idx_hbm.at[pl.ds(pg * _PG, _PG)], idx_v)

    @plsc.parallel_loop(0, _PG // 16, unroll=4)
    def _transpose(jj):
        idxv = idx_v[pl.ds(jj * 16, 16)]
        for d_ in range(_DG):
            row = jnp.full((16,), d_, jnp.int32)
            tv[d_, pl.ds(jj * 16, 16)] = plsc.load_gather(et_v, [row, idxv])

    for bb in range(_PG // 1024):
        pltpu.sync_copy(
            tv.at[:, pl.ds(bb * 1024, 1024)],
            out_hbm.at[pg * (_PG // 1024) + bb].at[pl.ds(dg * _DG, _DG), :],
        )


@functools.cache
def _gather():
    mesh = plsc.VectorSubcoreMesh(core_axis_name="c", subcore_axis_name="s")
    return pl.kernel(
        _gather_body,
        out_type=jax.ShapeDtypeStruct((16, D, 1024), jnp.float32),
        mesh=mesh,
        scratch_types=[
            pltpu.VMEM((_DG, K), jnp.float32),
            pltpu.VMEM((_PG,), jnp.int32),
            pltpu.VMEM((_DG, _PG), jnp.float32),
        ],
        compiler_params=pltpu.CompilerParams(needs_layout_passes=False),
    )


def kernel(z_e, embeddings):
    z_t = jnp.transpose(z_e, (0, 2, 1))
    idx3, loss_sum = _assign(z_t, embeddings)
    idx = idx3.reshape(ROWS)
    emb_t = jnp.transpose(embeddings)
    zq_t = _gather()(emb_t, idx)
    z_q_st = jnp.transpose(zq_t, (0, 2, 1))
    vq_loss = loss_sum[0, 0] * ((1.0 + BETA) / float(ROWS * D))
    return z_q_st, vq_loss
